# baseline (speedup 1.0000x reference)
import functools

import jax
import jax.numpy as jnp
import numpy as np
from jax import lax
from jax.experimental import pallas as pl
from jax.experimental.pallas import tpu as pltpu
from jax.experimental.pallas import tpu_sc as plsc

N = 10000
DEG = 32
K = 8
NB = 8
NCHUNK = N // NB
NWORK = 32
CPW = (NCHUNK + NWORK - 1) // NWORK
_LANES = 16

_DUP_PERM = np.array(list(range(8)) * 2, np.int32)


def _hi_perm(R):
    p = np.zeros(R // 2, np.int32)
    if R == 64:
        for l in range(16):
            for d in range(2):
                p[2 * l + d] = (4 + (l >> 3) + 2 * d) * 8 + (l & 7)
    else:
        for g in range(2):
            for l in range(16):
                for d in range(2):
                    p[32 * g + 2 * l + d] = (4 + 2 * g + d) * 16 + l
    return p


def _fold_weights(W, a_self, a_neigh, heads):
    eye = jnp.eye(heads, dtype=jnp.float32)
    An = jnp.kron(eye, a_neigh[:, None])
    As = jnp.kron(eye, a_self[:, None])
    R = W.shape[1]
    Sn = (W @ An)[:, _DUP_PERM]
    Sd = (W @ As)[:, _DUP_PERM]
    return jnp.concatenate(
        [W[:, :R // 2], Sn, Sd, W[:, R // 2 + _hi_perm(R)]], axis=1)


def _proj_body(R, x_ref, w_ref, ta_ref, sdup_ref, tb_ref):
    out = jnp.dot(x_ref[...], w_ref[...], preferred_element_type=jnp.float32)
    ta_ref[...] = out[:, :R // 2 + 16]
    sdup_ref[...] = out[:, R // 2 + 16:R // 2 + 32]
    tb_ref[...] = out[:, R // 2 + 32:].astype(jnp.bfloat16)


def _project(x, Wfull, bn=2000):
    n, d = x.shape
    R = Wfull.shape[1] - 32
    grid = (n + bn - 1) // bn
    return pl.pallas_call(
        functools.partial(_proj_body, R),
        grid=(grid,),
        in_specs=[
            pl.BlockSpec((bn, d), lambda i: (i, 0)),
            pl.BlockSpec((d, R + 32), lambda i: (0, 0)),
        ],
        out_specs=[
            pl.BlockSpec((bn, R // 2 + 16), lambda i: (i, 0)),
            pl.BlockSpec((bn, 16), lambda i: (i, 0)),
            pl.BlockSpec((bn, R // 2), lambda i: (i, 0)),
        ],
        out_shape=[
            jax.ShapeDtypeStruct((n, R // 2 + 16), jnp.float32),
            jax.ShapeDtypeStruct((n, 16), jnp.float32),
            jax.ShapeDtypeStruct((n, R // 2), jnp.bfloat16),
        ],
    )(x, Wfull)


def _bcast_lane(vec, idxv):
    dnums = lax.GatherDimensionNumbers(
        offset_dims=(), collapsed_slice_dims=(0,), start_index_map=(0,))
    return lax.gather(vec, idxv[:, None], dnums, slice_sizes=(1,),
                      mode=lax.GatherScatterMode.PROMISE_IN_BOUNDS)


def _leaky(e):
    return jnp.maximum(e, 0.01 * e)


def _make_sc_attention(R, final_layer):
    out_dim = 16 if final_layer else R
    acols = R // 2 + 16
    bcols = R // 2
    nacc = R // _LANES
    mesh = plsc.VectorSubcoreMesh(core_axis_name="c", subcore_axis_name="s")

    @functools.partial(
        pl.kernel,
        mesh=mesh,
        compiler_params=pltpu.CompilerParams(
            use_tc_tiling_on_sc=False, needs_layout_passes=False),
        out_type=jax.ShapeDtypeStruct((N, out_dim), jnp.float32),
        scratch_types=[
            pltpu.VMEM((CPW, 2, 128), jnp.int32),
            pltpu.VMEM((CPW * NB, 16), jnp.float32),
            pltpu.VMEM((CPW * NB, out_dim), jnp.float32),
            pltpu.VMEM((128, acols), jnp.float32),
            pltpu.VMEM((128, acols), jnp.float32),
            pltpu.VMEM((128, acols), jnp.float32),
            pltpu.VMEM((128, acols), jnp.float32),
            pltpu.VMEM((128, bcols), jnp.bfloat16),
            pltpu.VMEM((128, bcols), jnp.bfloat16),
            pltpu.VMEM((128, bcols), jnp.bfloat16),
            pltpu.VMEM((128, bcols), jnp.bfloat16),
            pltpu.SemaphoreType.DMA,
            pltpu.SemaphoreType.DMA,
            pltpu.SemaphoreType.DMA,
            pltpu.SemaphoreType.DMA,
            pltpu.SemaphoreType.DMA,
            pltpu.SemaphoreType.DMA,
            pltpu.SemaphoreType.DMA,
            pltpu.SemaphoreType.DMA,
        ],
    )
    def sc_attn(ta_hbm, sdup_hbm, tb_hbm, nbr_hbm, out_hbm,
                idx_all, sdup_v, out_v, ra0, ra1, rb0, rb1,
                qa0, qa1, qb0, qb1,
                sa0, sa1, sb0, sb1, za0, za1, zb0, zb1):
        wid = lax.axis_index("s") * 2 + lax.axis_index("c")
        lane = lax.iota(jnp.int32, 16)
        start = jnp.minimum(wid * CPW, NCHUNK - CPW)

        pltpu.sync_copy(nbr_hbm.at[pl.ds(start, CPW)], idx_all)
        pltpu.sync_copy(sdup_hbm.at[pl.ds(start * NB, CPW * NB)], sdup_v)

        def fire(c_local, r0, r1, q0, q1, s0, s1, z0, z1):
            i0 = idx_all.at[c_local, 0]
            i1 = idx_all.at[c_local, 1]
            return (pltpu.async_copy(ta_hbm.at[i0], r0, s0),
                    pltpu.async_copy(ta_hbm.at[i1], r1, s1),
                    pltpu.async_copy(tb_hbm.at[i0], q0, z0),
                    pltpu.async_copy(tb_hbm.at[i1], q1, z1))

        if final_layer:
            bidx = [jnp.full((16,), h, jnp.int32) for h in range(K)]
        else:
            bidx = [2 * j + (lane >> 3) for j in range(nacc)]

        zero = jnp.zeros((16,), jnp.float32)

        def compute(c_local, r0, r1, q0, q1):
            for half, rows, rowsb in ((0, r0, q0), (1, r1, q1)):

                def node_body(i, _):
                    nrow = c_local * NB + half * (NB // 2) + i
                    sself = sdup_v[nrow, :]
                    base = i * DEG
                    ssum = zero
                    acc = [zero] * nacc
                    for d in range(DEG):
                        j = base + d
                        srow = rows[j, pl.ds(R // 2, 16)]
                        ex = jnp.exp(_leaky(sself + srow))
                        ssum = ssum + ex
                        for r in range(nacc // 2):
                            w = _bcast_lane(ex, bidx[r])
                            acc[r] = acc[r] + w * rows[j, pl.ds(r * 16, 16)]
                        for g in range(nacc // 4):
                            fa, fb = plsc.unpack(
                                rowsb[j, pl.ds(32 * g, 32)],
                                format=plsc.PackFormat.INTERLEAVED)
                            ha = nacc // 2 + 2 * g
                            wa = _bcast_lane(ex, bidx[ha])
                            wb = _bcast_lane(ex, bidx[ha + 1])
                            acc[ha] = acc[ha] + wa * fa
                            acc[ha + 1] = acc[ha + 1] + wb * fb
                    rs = 1.0 / ssum
                    if final_layer:
                        msum = zero
                        for h in range(K):
                            msum = msum + _bcast_lane(rs, bidx[h]) * acc[h]
                        msum = msum * (1.0 / K)
                        ex = jnp.exp(msum)
                        cs = plsc.cumsum(ex)
                        totv = _bcast_lane(cs, jnp.full((16,), 15, jnp.int32))
                        out_v[nrow, :] = ex / totv
                    else:
                        for r in range(nacc):
                            o = _bcast_lane(rs, bidx[r]) * acc[r]
                            o = jnp.where(o > 0, o,
                                          jnp.exp(jnp.minimum(o, 0.0)) - 1.0)
                            out_v[nrow, pl.ds(r * 16, 16)] = o
                    return 0

                lax.fori_loop(0, NB // 2, node_body, 0)

        fire(0, ra0, ra1, qa0, qa1, sa0, sa1, za0, za1)

        def wait_a():
            i0 = idx_all.at[0, 0]
            pltpu.make_async_copy(ta_hbm.at[i0], ra0, sa0).wait()
            pltpu.make_async_copy(ta_hbm.at[i0], ra1, sa1).wait()
            pltpu.make_async_copy(tb_hbm.at[i0], qa0, za0).wait()
            pltpu.make_async_copy(tb_hbm.at[i0], qa1, za1).wait()

        def kbody2(kk, carry):
            k = 2 * kk
            hb = fire(k + 1, rb0, rb1, qb0, qb1, sb0, sb1, zb0, zb1)
            wait_a()
            compute(k, ra0, ra1, qa0, qa1)
            knext = jnp.minimum(k + 2, CPW - 1)
            fire(knext, ra0, ra1, qa0, qa1, sa0, sa1, za0, za1)
            for h in hb:
                h.wait()
            compute(k + 1, rb0, rb1, qb0, qb1)
            return carry

        lax.fori_loop(0, CPW // 2, kbody2, 0)
        wait_a()

        pltpu.sync_copy(out_v, out_hbm.at[pl.ds(start * NB, CPW * NB)])

    return sc_attn


_sc_attn1 = _make_sc_attention(64, final_layer=False)
_sc_attn2 = _make_sc_attention(128, final_layer=True)


def kernel(node_features, neighbors, W1, a1_self, a1_neigh, W2, a2_self, a2_neigh):
    nbr3 = neighbors.astype(jnp.int32).reshape(NCHUNK, 2, 128)
    Wf1 = _fold_weights(W1, a1_self, a1_neigh, K)
    Wf2 = _fold_weights(W2, a2_self, a2_neigh, K)

    t1a, sdup1, t1b = _project(node_features, Wf1)
    x1 = _sc_attn1(t1a, sdup1, t1b, nbr3)
    t2a, sdup2, t2b = _project(x1, Wf2)
    return _sc_attn2(t2a, sdup2, t2b, nbr3)
